# baseline (speedup 1.0000x reference)
import functools

import jax
import jax.numpy as jnp
from jax import lax
from jax.experimental import pallas as pl
from jax.experimental.pallas import tpu as pltpu
from jax.experimental.pallas import tpu_sc as plsc

_NC = 2
_NS = 16
_NW = _NC * _NS


def _make_sc_gather(V, D, B, C):
    n_per_w = B // _NW
    n_iter = n_per_w // C
    assert n_per_w % C == 0 and C % 8 == 0

    mesh = plsc.VectorSubcoreMesh(core_axis_name="c", subcore_axis_name="s")

    @functools.partial(
        pl.kernel,
        mesh=mesh,
        out_type=jax.ShapeDtypeStruct((B, D), jnp.float32),
        scratch_types=[
            pltpu.VMEM((C,), jnp.int32),
            pltpu.VMEM((C, D), jnp.float32),
            pltpu.SemaphoreType.DMA,
        ],
        compiler_params=pltpu.CompilerParams(use_tc_tiling_on_sc=False),
    )
    def gather(table_hbm, idx_hbm, out_hbm, idx_v, rows_v, sem):
        wid = lax.axis_index("s") * _NC + lax.axis_index("c")
        for i in range(n_iter):
            base = wid * n_per_w + i * C
            pltpu.sync_copy(idx_hbm.at[pl.ds(base, C)], idx_v)
            pltpu.async_copy(table_hbm.at[idx_v], rows_v, sem).wait()
            pltpu.sync_copy(rows_v, out_hbm.at[pl.ds(base, C)])

    return gather


def _dotT(a, b):
    return lax.dot_general(a, b, (((1,), (1,)), ((), ())),
                           preferred_element_type=jnp.float32)


def _dot(a, b):
    return jnp.dot(a, b, preferred_element_type=jnp.float32)


def _mlp_loss_body(n_total, g_ref, t2_ref, te_ref, wo1a_ref, wo1b_ref,
                   wi1a_ref, wi1b_ref, wo2_ref, wi2_ref, bo1_ref, bi1_ref,
                   bo2_ref, bi2_ref, out_ref):
    i = pl.program_id(0)
    nb = pl.num_programs(0)
    D = te_ref.shape[0]
    bt = g_ref.shape[2]
    f32 = jnp.float32

    te_t = te_ref[...]
    tb_out_t = _dot(wo1b_ref[...], te_t) + bo1_ref[...]
    tb_in_t = _dot(wi1b_ref[...], te_t) + bi1_ref[...]

    hd = t2_ref[...] % 24
    seg = jnp.where((hd >= 22) | (hd < 6), 0, jnp.where(hd < 14, 1, 2))
    io3 = lax.broadcasted_iota(jnp.int32, (3, bt), 0)
    selx = (io3 == seg[0:1]).astype(f32)
    sely = (io3 == seg[1:2]).astype(f32)

    xg_t = g_ref[0]
    yg_t = g_ref[1]
    ng_t = g_ref[2]

    hx = jnp.maximum(_dot(wo1a_ref[...], xg_t) + _dot(tb_out_t, selx), 0.0)
    hy = jnp.maximum(_dot(wi1a_ref[...], yg_t) + _dot(tb_in_t, sely), 0.0)
    hn = jnp.maximum(_dot(wi1a_ref[...], ng_t) + tb_in_t[:, 0:1], 0.0)
    xi_x = _dot(wo2_ref[...], hx) + bo2_ref[...]
    xi_y = _dot(wi2_ref[...], hy) + bi2_ref[...]
    xi_n = _dot(wi2_ref[...], hn) + bi2_ref[...]

    dp = xi_x - xi_y
    dn = xi_x - xi_n
    ones = jnp.ones((1, D), f32)
    pd = jnp.sqrt(_dot(ones, dp * dp))
    nd = jnp.sqrt(_dot(ones, dn * dn))
    zd = nd - pd
    ls = jnp.minimum(zd, 0.0) - jnp.log1p(jnp.exp(-jnp.abs(zd)))
    partial = jnp.sum(ls, keepdims=True).reshape(1, 1)

    @pl.when(i == 0)
    def _init():
        out_ref[...] = jnp.zeros_like(out_ref)

    out_ref[...] += partial

    @pl.when(i == nb - 1)
    def _finish():
        out_ref[...] = out_ref[...] * (-1.0 / n_total)


def _mlp_loss(g, t2, te_t, wo1a_t, wo1b_t, wi1a_t, wi1b_t, wo2_t, wi2_t,
              bo1_t, bi1_t, bo2_t, bi2_t, bt):
    n = g.shape[2]
    grid = (n // bt,)
    full = lambda s: pl.BlockSpec(s, lambda i: tuple(0 for _ in s))
    return pl.pallas_call(
        functools.partial(_mlp_loss_body, n),
        grid=grid,
        in_specs=[
            pl.BlockSpec((3, g.shape[1], bt), lambda i: (0, 0, i)),
            pl.BlockSpec((2, bt), lambda i: (0, i)),
            full(te_t.shape),
            full(wo1a_t.shape), full(wo1b_t.shape),
            full(wi1a_t.shape), full(wi1b_t.shape),
            full(wo2_t.shape), full(wi2_t.shape),
            full(bo1_t.shape), full(bi1_t.shape),
            full(bo2_t.shape), full(bi2_t.shape),
        ],
        out_specs=pl.BlockSpec((1, 1), lambda i: (0, 0)),
        out_shape=jax.ShapeDtypeStruct((1, 1), jnp.float32),
    )(g, t2, te_t, wo1a_t, wo1b_t, wi1a_t, wi1b_t, wo2_t, wi2_t,
      bo1_t, bi1_t, bo2_t, bi2_t)


def kernel(x, x_t_slot, y, y_t_slot, vecs_use, time_embeddings,
           W_out1, b_out1, W_out2, b_out2, W_in1, b_in1, W_in2, b_in2):
    seq_len, user_len = x.shape
    n = seq_len * user_len
    v, d = vecs_use.shape

    neg_idx = jax.random.randint(jax.random.key(1234), (n,), 0, v, dtype=jnp.int32)
    idx_all = jnp.concatenate([x.reshape(-1), y.reshape(-1), neg_idx])

    g = _make_sc_gather(v, d, 3 * n, 4800)(vecs_use, idx_all)
    g = g.reshape(3, n, d).transpose(0, 2, 1)

    t2 = jnp.stack([x_t_slot.reshape(-1), y_t_slot.reshape(-1)], axis=0)

    loss = _mlp_loss(
        g, t2,
        time_embeddings.T,
        W_out1[:d].T, W_out1[d:].T,
        W_in1[:d].T, W_in1[d:].T,
        W_out2.T, W_in2.T,
        b_out1.reshape(d, 1), b_in1.reshape(d, 1),
        b_out2.reshape(d, 1), b_in2.reshape(d, 1),
        bt=8192,
    )
    return loss.reshape(())


# speedup vs baseline: 1.3817x; 1.3817x over previous
import functools

import jax
import jax.numpy as jnp
from jax import lax
from jax.experimental import pallas as pl
from jax.experimental.pallas import tpu as pltpu
from jax.experimental.pallas import tpu_sc as plsc

_NC = 2
_NS = 16
_NW = _NC * _NS


def _make_sc_gather(V, D, B, C):
    n_per_w = B // _NW
    n_iter = n_per_w // C
    assert n_per_w % C == 0 and C % 8 == 0 and n_iter % 2 == 0

    mesh = plsc.VectorSubcoreMesh(core_axis_name="c", subcore_axis_name="s")

    @functools.partial(
        pl.kernel,
        mesh=mesh,
        out_type=jax.ShapeDtypeStruct((B, D), jnp.float32),
        scratch_types=[
            pltpu.VMEM((2, C), jnp.int32),
            pltpu.VMEM((2, C, D), jnp.float32),
            pltpu.SemaphoreType.DMA,
            pltpu.SemaphoreType.DMA,
            pltpu.SemaphoreType.DMA,
            pltpu.SemaphoreType.DMA,
            pltpu.SemaphoreType.DMA,
            pltpu.SemaphoreType.DMA,
        ],
        compiler_params=pltpu.CompilerParams(use_tc_tiling_on_sc=False),
    )
    def gather(table_hbm, idx_hbm, out_hbm, idx_v, rows_v,
               isem0, isem1, gsem0, gsem1, osem0, osem1):
        wid = lax.axis_index("s") * _NC + lax.axis_index("c")
        isems = (isem0, isem1)
        gsems = (gsem0, gsem1)
        osems = (osem0, osem1)

        def base_of(i):
            return wid * n_per_w + i * C

        def idx_cp(i, s):
            return pltpu.make_async_copy(
                idx_hbm.at[pl.ds(base_of(i), C)], idx_v.at[s], isems[s])

        def gat_cp(s):
            return pltpu.make_async_copy(
                table_hbm.at[idx_v.at[s]], rows_v.at[s], gsems[s])

        def out_cp(i, s):
            return pltpu.make_async_copy(
                rows_v.at[s], out_hbm.at[pl.ds(base_of(i), C)], osems[s])

        idx_cp(0, 0).start()
        idx_cp(1, 1).start()
        for i in range(n_iter):
            s = i % 2
            idx_cp(i, s).wait()
            if i >= 2:
                out_cp(i - 2, s).wait()
            gat_cp(s).start()
            gat_cp(s).wait()
            out_cp(i, s).start()
            if i + 2 < n_iter:
                idx_cp(i + 2, s).start()
        if n_iter >= 2:
            out_cp(n_iter - 2, (n_iter - 2) % 2).wait()
        out_cp(n_iter - 1, (n_iter - 1) % 2).wait()

    return gather


def _dotT(a, b):
    return lax.dot_general(a, b, (((1,), (1,)), ((), ())),
                           preferred_element_type=jnp.float32)


def _dot(a, b):
    return jnp.dot(a, b, preferred_element_type=jnp.float32)


def _mlp_loss_body(n_total, g_ref, t2_ref, te_ref, wo1a_ref, wo1b_ref,
                   wi1a_ref, wi1b_ref, wo2_ref, wi2_ref, bo1_ref, bi1_ref,
                   bo2_ref, bi2_ref, out_ref):
    i = pl.program_id(0)
    nb = pl.num_programs(0)
    D = te_ref.shape[0]
    bt = g_ref.shape[1]
    f32 = jnp.float32

    te_t = te_ref[...]
    tb_out_t = _dot(wo1b_ref[...], te_t) + bo1_ref[...]
    tb_in_t = _dot(wi1b_ref[...], te_t) + bi1_ref[...]

    hd = t2_ref[...] % 24
    seg = jnp.where((hd >= 22) | (hd < 6), 0, jnp.where(hd < 14, 1, 2))
    io3 = lax.broadcasted_iota(jnp.int32, (3, bt), 0)
    selx = (io3 == seg[0:1]).astype(f32)
    sely = (io3 == seg[1:2]).astype(f32)

    xg = g_ref[0]
    yg = g_ref[1]
    ng = g_ref[2]

    hx = jnp.maximum(_dotT(wo1a_ref[...], xg) + _dot(tb_out_t, selx), 0.0)
    hy = jnp.maximum(_dotT(wi1a_ref[...], yg) + _dot(tb_in_t, sely), 0.0)
    hn = jnp.maximum(_dotT(wi1a_ref[...], ng) + tb_in_t[:, 0:1], 0.0)
    xi_x = _dot(wo2_ref[...], hx) + bo2_ref[...]
    xi_y = _dot(wi2_ref[...], hy) + bi2_ref[...]
    xi_n = _dot(wi2_ref[...], hn) + bi2_ref[...]

    dp = xi_x - xi_y
    dn = xi_x - xi_n
    ones = jnp.ones((1, D), f32)
    pd = jnp.sqrt(_dot(ones, dp * dp))
    nd = jnp.sqrt(_dot(ones, dn * dn))
    zd = nd - pd
    ls = jnp.minimum(zd, 0.0) - jnp.log1p(jnp.exp(-jnp.abs(zd)))
    partial = jnp.sum(ls, keepdims=True).reshape(1, 1)

    @pl.when(i == 0)
    def _init():
        out_ref[...] = jnp.zeros_like(out_ref)

    out_ref[...] += partial

    @pl.when(i == nb - 1)
    def _finish():
        out_ref[...] = out_ref[...] * (-1.0 / n_total)


def _mlp_loss(g, t2, te_t, wo1a_t, wo1b_t, wi1a_t, wi1b_t, wo2_t, wi2_t,
              bo1_t, bi1_t, bo2_t, bi2_t, bt):
    n = g.shape[1]
    grid = (n // bt,)
    full = lambda s: pl.BlockSpec(s, lambda i: tuple(0 for _ in s))
    return pl.pallas_call(
        functools.partial(_mlp_loss_body, n),
        grid=grid,
        in_specs=[
            pl.BlockSpec((3, bt, g.shape[2]), lambda i: (0, i, 0)),
            pl.BlockSpec((2, bt), lambda i: (0, i)),
            full(te_t.shape),
            full(wo1a_t.shape), full(wo1b_t.shape),
            full(wi1a_t.shape), full(wi1b_t.shape),
            full(wo2_t.shape), full(wi2_t.shape),
            full(bo1_t.shape), full(bi1_t.shape),
            full(bo2_t.shape), full(bi2_t.shape),
        ],
        out_specs=pl.BlockSpec((1, 1), lambda i: (0, 0)),
        out_shape=jax.ShapeDtypeStruct((1, 1), jnp.float32),
    )(g, t2, te_t, wo1a_t, wo1b_t, wi1a_t, wi1b_t, wo2_t, wi2_t,
      bo1_t, bi1_t, bo2_t, bi2_t)


def kernel(x, x_t_slot, y, y_t_slot, vecs_use, time_embeddings,
           W_out1, b_out1, W_out2, b_out2, W_in1, b_in1, W_in2, b_in2):
    seq_len, user_len = x.shape
    n = seq_len * user_len
    v, d = vecs_use.shape

    neg_idx = jax.random.randint(jax.random.key(1234), (n,), 0, v, dtype=jnp.int32)
    idx_all = jnp.concatenate([x.reshape(-1), y.reshape(-1), neg_idx])

    g = _make_sc_gather(v, d, 3 * n, 2400)(vecs_use, idx_all)
    g = g.reshape(3, n, d)

    t2 = jnp.stack([x_t_slot.reshape(-1), y_t_slot.reshape(-1)], axis=0)

    loss = _mlp_loss(
        g, t2,
        time_embeddings.T,
        W_out1[:d].T, W_out1[d:].T,
        W_in1[:d].T, W_in1[d:].T,
        W_out2.T, W_in2.T,
        b_out1.reshape(d, 1), b_in1.reshape(d, 1),
        b_out2.reshape(d, 1), b_in2.reshape(d, 1),
        bt=4096,
    )
    return loss.reshape(())


# speedup vs baseline: 1.4511x; 1.0502x over previous
import functools

import jax
import jax.numpy as jnp
from jax import lax
from jax.experimental import pallas as pl
from jax.experimental.pallas import tpu as pltpu
from jax.experimental.pallas import tpu_sc as plsc

_NC = 2
_NS = 16
_NW = _NC * _NS


def _make_sc_gather(V, D, B, C):
    n_per_w = B // _NW
    n_iter = n_per_w // C
    assert n_per_w % C == 0 and C % 8 == 0 and n_iter % 2 == 0

    mesh = plsc.VectorSubcoreMesh(core_axis_name="c", subcore_axis_name="s")

    @functools.partial(
        pl.kernel,
        mesh=mesh,
        out_type=jax.ShapeDtypeStruct((B, D), jnp.float32),
        scratch_types=[
            pltpu.VMEM((2, C), jnp.int32),
            pltpu.VMEM((2, C, D), jnp.float32),
            pltpu.SemaphoreType.DMA,
            pltpu.SemaphoreType.DMA,
            pltpu.SemaphoreType.DMA,
            pltpu.SemaphoreType.DMA,
            pltpu.SemaphoreType.DMA,
            pltpu.SemaphoreType.DMA,
        ],
        compiler_params=pltpu.CompilerParams(use_tc_tiling_on_sc=False),
    )
    def gather(table_hbm, idx_hbm, out_hbm, idx_v, rows_v,
               isem0, isem1, gsem0, gsem1, osem0, osem1):
        wid = lax.axis_index("s") * _NC + lax.axis_index("c")
        isems = (isem0, isem1)
        gsems = (gsem0, gsem1)
        osems = (osem0, osem1)

        def base_of(i):
            return wid * n_per_w + i * C

        def idx_cp(i, s):
            return pltpu.make_async_copy(
                idx_hbm.at[pl.ds(base_of(i), C)], idx_v.at[s], isems[s])

        def gat_cp(s):
            return pltpu.make_async_copy(
                table_hbm.at[idx_v.at[s]], rows_v.at[s], gsems[s])

        def out_cp(i, s):
            return pltpu.make_async_copy(
                rows_v.at[s], out_hbm.at[pl.ds(base_of(i), C)], osems[s])

        idx_cp(0, 0).start()
        idx_cp(1, 1).start()
        for i in range(n_iter):
            s = i % 2
            idx_cp(i, s).wait()
            if i >= 2:
                out_cp(i - 2, s).wait()
            gat_cp(s).start()
            gat_cp(s).wait()
            out_cp(i, s).start()
            if i + 2 < n_iter:
                idx_cp(i + 2, s).start()
        if n_iter >= 2:
            out_cp(n_iter - 2, (n_iter - 2) % 2).wait()
        out_cp(n_iter - 1, (n_iter - 1) % 2).wait()

    return gather


def _dotT(a, b):
    return lax.dot_general(a, b, (((1,), (1,)), ((), ())),
                           preferred_element_type=jnp.float32)


def _dot(a, b):
    return jnp.dot(a, b, preferred_element_type=jnp.float32)


def _mlp_loss_body(n_total, g_ref, t2_ref, te_ref, wo1a_ref, wo1b_ref,
                   wi1a_ref, wi1b_ref, wo2_ref, wi2_ref, bo1_ref, bi1_ref,
                   bo2_ref, bi2_ref, out_ref):
    i = pl.program_id(0)
    nb = pl.num_programs(0)
    D = te_ref.shape[0]
    bt = g_ref.shape[1]
    f32 = jnp.float32

    te_t = te_ref[...]
    tb_out_t = _dot(wo1b_ref[...], te_t) + bo1_ref[...]
    tb_in_t = _dot(wi1b_ref[...], te_t) + bi1_ref[...]

    hd = t2_ref[...] % 24
    seg = jnp.where((hd >= 22) | (hd < 6), 0, jnp.where(hd < 14, 1, 2))
    io3 = lax.broadcasted_iota(jnp.int32, (3, bt), 0)
    selx = (io3 == seg[0:1]).astype(f32)
    sely = (io3 == seg[1:2]).astype(f32)

    xg = g_ref[0]
    yg = g_ref[1]
    ng = g_ref[2]

    hx = jnp.maximum(_dotT(wo1a_ref[...], xg) + _dot(tb_out_t, selx), 0.0)
    hy = jnp.maximum(_dotT(wi1a_ref[...], yg) + _dot(tb_in_t, sely), 0.0)
    hn = jnp.maximum(_dotT(wi1a_ref[...], ng) + tb_in_t[:, 0:1], 0.0)
    xi_x = _dot(wo2_ref[...], hx) + bo2_ref[...]
    xi_y = _dot(wi2_ref[...], hy) + bi2_ref[...]
    xi_n = _dot(wi2_ref[...], hn) + bi2_ref[...]

    dp = xi_x - xi_y
    dn = xi_x - xi_n
    ones = jnp.ones((1, D), f32)
    pd = jnp.sqrt(_dot(ones, dp * dp))
    nd = jnp.sqrt(_dot(ones, dn * dn))
    zd = nd - pd
    ls = jnp.minimum(zd, 0.0) - jnp.log1p(jnp.exp(-jnp.abs(zd)))
    partial = jnp.sum(ls, keepdims=True).reshape(1, 1)

    @pl.when(i == 0)
    def _init():
        out_ref[...] = jnp.zeros_like(out_ref)

    out_ref[...] += partial

    @pl.when(i == nb - 1)
    def _finish():
        out_ref[...] = out_ref[...] * (-1.0 / n_total)


def _mlp_loss(g, t2, te_t, wo1a_t, wo1b_t, wi1a_t, wi1b_t, wo2_t, wi2_t,
              bo1_t, bi1_t, bo2_t, bi2_t, bt):
    n = g.shape[1]
    grid = (n // bt,)
    full = lambda s: pl.BlockSpec(s, lambda i: tuple(0 for _ in s))
    return pl.pallas_call(
        functools.partial(_mlp_loss_body, n),
        grid=grid,
        in_specs=[
            pl.BlockSpec((3, bt, g.shape[2]), lambda i: (0, i, 0)),
            pl.BlockSpec((2, bt), lambda i: (0, i)),
            full(te_t.shape),
            full(wo1a_t.shape), full(wo1b_t.shape),
            full(wi1a_t.shape), full(wi1b_t.shape),
            full(wo2_t.shape), full(wi2_t.shape),
            full(bo1_t.shape), full(bi1_t.shape),
            full(bo2_t.shape), full(bi2_t.shape),
        ],
        out_specs=pl.BlockSpec((1, 1), lambda i: (0, 0)),
        out_shape=jax.ShapeDtypeStruct((1, 1), jnp.float32),
    )(g, t2, te_t, wo1a_t, wo1b_t, wi1a_t, wi1b_t, wo2_t, wi2_t,
      bo1_t, bi1_t, bo2_t, bi2_t)


def kernel(x, x_t_slot, y, y_t_slot, vecs_use, time_embeddings,
           W_out1, b_out1, W_out2, b_out2, W_in1, b_in1, W_in2, b_in2):
    seq_len, user_len = x.shape
    n = seq_len * user_len
    v, d = vecs_use.shape

    neg_idx = jax.random.randint(jax.random.key(1234), (n,), 0, v, dtype=jnp.int32)
    idx_all = jnp.concatenate([x.reshape(-1), y.reshape(-1), neg_idx])

    g = _make_sc_gather(v, d, 3 * n, 2400)(vecs_use, idx_all)
    g = g.reshape(3, n, d)

    t2 = jnp.stack([x_t_slot.reshape(-1), y_t_slot.reshape(-1)], axis=0)

    loss = _mlp_loss(
        g, t2,
        time_embeddings.T,
        W_out1[:d].T, W_out1[d:].T,
        W_in1[:d].T, W_in1[d:].T,
        W_out2.T, W_in2.T,
        b_out1.reshape(d, 1), b_in1.reshape(d, 1),
        b_out2.reshape(d, 1), b_in2.reshape(d, 1),
        bt=8192,
    )
    return loss.reshape(())


# speedup vs baseline: 1.4738x; 1.0157x over previous
import functools

import jax
import jax.numpy as jnp
from jax import lax
from jax.experimental import pallas as pl
from jax.experimental.pallas import tpu as pltpu
from jax.experimental.pallas import tpu_sc as plsc

_NC = 2
_NS = 16
_NW = _NC * _NS


def _make_sc_gather(V, D, B, C):
    n_per_w = B // _NW
    n_iter = n_per_w // C
    assert n_per_w % C == 0 and C % 8 == 0 and n_iter % 2 == 0

    mesh = plsc.VectorSubcoreMesh(core_axis_name="c", subcore_axis_name="s")

    @functools.partial(
        pl.kernel,
        mesh=mesh,
        out_type=jax.ShapeDtypeStruct((B, D), jnp.float32),
        scratch_types=[
            pltpu.VMEM((2, C), jnp.int32),
            pltpu.VMEM((2, C, D), jnp.float32),
            pltpu.SemaphoreType.DMA,
            pltpu.SemaphoreType.DMA,
            pltpu.SemaphoreType.DMA,
            pltpu.SemaphoreType.DMA,
            pltpu.SemaphoreType.DMA,
            pltpu.SemaphoreType.DMA,
        ],
        compiler_params=pltpu.CompilerParams(use_tc_tiling_on_sc=False),
    )
    def gather(table_hbm, idx_hbm, out_hbm, idx_v, rows_v,
               isem0, isem1, gsem0, gsem1, osem0, osem1):
        wid = lax.axis_index("s") * _NC + lax.axis_index("c")
        isems = (isem0, isem1)
        gsems = (gsem0, gsem1)
        osems = (osem0, osem1)

        def base_of(i):
            return wid * n_per_w + i * C

        def idx_cp(i, s):
            return pltpu.make_async_copy(
                idx_hbm.at[pl.ds(base_of(i), C)], idx_v.at[s], isems[s])

        def gat_cp(s):
            return pltpu.make_async_copy(
                table_hbm.at[idx_v.at[s]], rows_v.at[s], gsems[s])

        def out_cp(i, s):
            return pltpu.make_async_copy(
                rows_v.at[s], out_hbm.at[pl.ds(base_of(i), C)], osems[s])

        idx_cp(0, 0).start()
        idx_cp(1, 1).start()
        for i in range(n_iter):
            s = i % 2
            idx_cp(i, s).wait()
            if i >= 2:
                out_cp(i - 2, s).wait()
            gat_cp(s).start()
            gat_cp(s).wait()
            out_cp(i, s).start()
            if i + 2 < n_iter:
                idx_cp(i + 2, s).start()
        if n_iter >= 2:
            out_cp(n_iter - 2, (n_iter - 2) % 2).wait()
        out_cp(n_iter - 1, (n_iter - 1) % 2).wait()

    return gather


def _dotT(a, b):
    return lax.dot_general(a, b, (((1,), (1,)), ((), ())),
                           preferred_element_type=jnp.float32)


def _dot(a, b):
    return jnp.dot(a, b, preferred_element_type=jnp.float32)


def _mlp_loss_body(n_total, g_ref, t2_ref, te_ref, wo1a_ref, wo1b_ref,
                   wi1a_ref, wi1b_ref, wo2_ref, wi2_ref, bo1_ref, bi1_ref,
                   bo2_ref, bi2_ref, out_ref):
    i = pl.program_id(0)
    nb = pl.num_programs(0)
    D = te_ref.shape[0]
    bt = g_ref.shape[1]
    f32 = jnp.float32

    te_t = te_ref[...]
    tb_out_t = _dot(wo1b_ref[...], te_t) + bo1_ref[...]
    tb_in_t = _dot(wi1b_ref[...], te_t) + bi1_ref[...]

    hd = t2_ref[...] % 24
    seg = jnp.where((hd >= 22) | (hd < 6), 0, jnp.where(hd < 14, 1, 2))
    io3 = lax.broadcasted_iota(jnp.int32, (3, bt), 0)
    selx = (io3 == seg[0:1]).astype(f32)
    sely = (io3 == seg[1:2]).astype(f32)

    xg = g_ref[0]
    yg = g_ref[1]
    ng = g_ref[2]

    hx = jnp.maximum(_dotT(wo1a_ref[...], xg) + _dot(tb_out_t, selx), 0.0)
    hy = jnp.maximum(_dotT(wi1a_ref[...], yg) + _dot(tb_in_t, sely), 0.0)
    hn = jnp.maximum(_dotT(wi1a_ref[...], ng) + tb_in_t[:, 0:1], 0.0)
    xi_x = _dot(wo2_ref[...], hx) + bo2_ref[...]
    xi_y = _dot(wi2_ref[...], hy) + bi2_ref[...]
    xi_n = _dot(wi2_ref[...], hn) + bi2_ref[...]

    dp = xi_x - xi_y
    dn = xi_x - xi_n
    ones = jnp.ones((1, D), f32)
    pd = jnp.sqrt(_dot(ones, dp * dp))
    nd = jnp.sqrt(_dot(ones, dn * dn))
    zd = nd - pd
    ls = jnp.minimum(zd, 0.0) - jnp.log1p(jnp.exp(-jnp.abs(zd)))
    partial = jnp.sum(ls, keepdims=True).reshape(1, 1)

    @pl.when(i == 0)
    def _init():
        out_ref[...] = jnp.zeros_like(out_ref)

    out_ref[...] += partial

    @pl.when(i == nb - 1)
    def _finish():
        out_ref[...] = out_ref[...] * (-1.0 / n_total)


def _mlp_loss(g, t2, te_t, wo1a_t, wo1b_t, wi1a_t, wi1b_t, wo2_t, wi2_t,
              bo1_t, bi1_t, bo2_t, bi2_t, bt):
    n = g.shape[1]
    grid = (n // bt,)
    full = lambda s: pl.BlockSpec(s, lambda i: tuple(0 for _ in s))
    return pl.pallas_call(
        functools.partial(_mlp_loss_body, n),
        grid=grid,
        in_specs=[
            pl.BlockSpec((3, bt, g.shape[2]), lambda i: (0, i, 0)),
            pl.BlockSpec((2, bt), lambda i: (0, i)),
            full(te_t.shape),
            full(wo1a_t.shape), full(wo1b_t.shape),
            full(wi1a_t.shape), full(wi1b_t.shape),
            full(wo2_t.shape), full(wi2_t.shape),
            full(bo1_t.shape), full(bi1_t.shape),
            full(bo2_t.shape), full(bi2_t.shape),
        ],
        out_specs=pl.BlockSpec((1, 1), lambda i: (0, 0)),
        out_shape=jax.ShapeDtypeStruct((1, 1), jnp.float32),
    )(g, t2, te_t, wo1a_t, wo1b_t, wi1a_t, wi1b_t, wo2_t, wi2_t,
      bo1_t, bi1_t, bo2_t, bi2_t)


def kernel(x, x_t_slot, y, y_t_slot, vecs_use, time_embeddings,
           W_out1, b_out1, W_out2, b_out2, W_in1, b_in1, W_in2, b_in2):
    seq_len, user_len = x.shape
    n = seq_len * user_len
    v, d = vecs_use.shape

    neg_idx = jax.random.randint(jax.random.key(1234), (n,), 0, v, dtype=jnp.int32)
    idx_all = jnp.concatenate([x.reshape(-1), y.reshape(-1), neg_idx])

    g = _make_sc_gather(v, d, 3 * n, 2400)(vecs_use, idx_all)
    g = g.reshape(3, n, d)

    t2 = jnp.stack([x_t_slot.reshape(-1), y_t_slot.reshape(-1)], axis=0)

    loss = _mlp_loss(
        g, t2,
        time_embeddings.T,
        W_out1[:d].T, W_out1[d:].T,
        W_in1[:d].T, W_in1[d:].T,
        W_out2.T, W_in2.T,
        b_out1.reshape(d, 1), b_in1.reshape(d, 1),
        b_out2.reshape(d, 1), b_in2.reshape(d, 1),
        bt=16384,
    )
    return loss.reshape(())


# speedup vs baseline: 1.5002x; 1.0179x over previous
import functools

import jax
import jax.numpy as jnp
from jax import lax
from jax.experimental import pallas as pl
from jax.experimental.pallas import tpu as pltpu
from jax.experimental.pallas import tpu_sc as plsc

_NC = 2
_NS = 16
_NW = _NC * _NS


def _make_sc_gather(V, D, B, C):
    n_per_w = B // _NW
    n_iter = n_per_w // C
    assert n_per_w % C == 0 and C % 8 == 0 and n_iter % 2 == 0

    mesh = plsc.VectorSubcoreMesh(core_axis_name="c", subcore_axis_name="s")

    @functools.partial(
        pl.kernel,
        mesh=mesh,
        out_type=jax.ShapeDtypeStruct((B, D), jnp.float32),
        scratch_types=[
            pltpu.VMEM((2, C), jnp.int32),
            pltpu.VMEM((2, C, D), jnp.float32),
            pltpu.SemaphoreType.DMA,
            pltpu.SemaphoreType.DMA,
            pltpu.SemaphoreType.DMA,
            pltpu.SemaphoreType.DMA,
            pltpu.SemaphoreType.DMA,
            pltpu.SemaphoreType.DMA,
        ],
        compiler_params=pltpu.CompilerParams(use_tc_tiling_on_sc=False),
    )
    def gather(table_hbm, idx_hbm, out_hbm, idx_v, rows_v,
               isem0, isem1, gsem0, gsem1, osem0, osem1):
        wid = lax.axis_index("s") * _NC + lax.axis_index("c")
        isems = (isem0, isem1)
        gsems = (gsem0, gsem1)
        osems = (osem0, osem1)

        def base_of(i):
            return wid * n_per_w + i * C

        def idx_cp(i, s):
            return pltpu.make_async_copy(
                idx_hbm.at[pl.ds(base_of(i), C)], idx_v.at[s], isems[s])

        def gat_cp(s):
            return pltpu.make_async_copy(
                table_hbm.at[idx_v.at[s]], rows_v.at[s], gsems[s])

        def out_cp(i, s):
            return pltpu.make_async_copy(
                rows_v.at[s], out_hbm.at[pl.ds(base_of(i), C)], osems[s])

        idx_cp(0, 0).start()
        idx_cp(1, 1).start()
        for i in range(n_iter):
            s = i % 2
            idx_cp(i, s).wait()
            if i >= 2:
                out_cp(i - 2, s).wait()
            gat_cp(s).start()
            gat_cp(s).wait()
            out_cp(i, s).start()
            if i + 2 < n_iter:
                idx_cp(i + 2, s).start()
        if n_iter >= 2:
            out_cp(n_iter - 2, (n_iter - 2) % 2).wait()
        out_cp(n_iter - 1, (n_iter - 1) % 2).wait()

    return gather


def _dotT(a, b):
    return lax.dot_general(a, b, (((1,), (1,)), ((), ())),
                           preferred_element_type=jnp.float32)


def _dot(a, b):
    return jnp.dot(a, b, preferred_element_type=jnp.float32)


def _mlp_loss_body(n_total, g_ref, t2_ref, te_ref, wo1a_ref, wo1b_ref,
                   wi1a_ref, wi1b_ref, wo2_ref, wi2_ref, bo1_ref, bi1_ref,
                   bo2_ref, bi2_ref, out_ref):
    i = pl.program_id(0)
    nb = pl.num_programs(0)
    D = te_ref.shape[0]
    bt = g_ref.shape[1]
    f32 = jnp.float32

    te_t = te_ref[...]
    tb_out_t = _dot(wo1b_ref[...], te_t) + bo1_ref[...]
    tb_in_t = _dot(wi1b_ref[...], te_t) + bi1_ref[...]

    hd = t2_ref[...] % 24
    seg = jnp.where((hd >= 22) | (hd < 6), 0, jnp.where(hd < 14, 1, 2))
    io3 = lax.broadcasted_iota(jnp.int32, (3, bt), 0)
    selx = (io3 == seg[0:1]).astype(f32)
    sely = (io3 == seg[1:2]).astype(f32)

    xg = g_ref[0]
    yg = g_ref[1]
    ng = g_ref[2]

    hx = jnp.maximum(_dotT(wo1a_ref[...], xg) + _dot(tb_out_t, selx), 0.0)
    hy = jnp.maximum(_dotT(wi1a_ref[...], yg) + _dot(tb_in_t, sely), 0.0)
    hn = jnp.maximum(_dotT(wi1a_ref[...], ng) + tb_in_t[:, 0:1], 0.0)
    xi_x = _dot(wo2_ref[...], hx) + bo2_ref[...]
    xi_y = _dot(wi2_ref[...], hy) + bi2_ref[...]
    xi_n = _dot(wi2_ref[...], hn) + bi2_ref[...]

    dp = xi_x - xi_y
    dn = xi_x - xi_n
    ones = jnp.ones((1, D), f32)
    pd = jnp.sqrt(_dot(ones, dp * dp))
    nd = jnp.sqrt(_dot(ones, dn * dn))
    zd = nd - pd
    ls = jnp.minimum(zd, 0.0) - jnp.log1p(jnp.exp(-jnp.abs(zd)))
    partial = jnp.sum(ls, keepdims=True).reshape(1, 1)

    @pl.when(i == 0)
    def _init():
        out_ref[...] = jnp.zeros_like(out_ref)

    out_ref[...] += partial

    @pl.when(i == nb - 1)
    def _finish():
        out_ref[...] = out_ref[...] * (-1.0 / n_total)


def _mlp_loss(g, t2, te_t, wo1a_t, wo1b_t, wi1a_t, wi1b_t, wo2_t, wi2_t,
              bo1_t, bi1_t, bo2_t, bi2_t, bt, total_n):
    n = g.shape[1]
    grid = (n // bt,)
    full = lambda s: pl.BlockSpec(s, lambda i: tuple(0 for _ in s))
    return pl.pallas_call(
        functools.partial(_mlp_loss_body, total_n),
        grid=grid,
        in_specs=[
            pl.BlockSpec((3, bt, g.shape[2]), lambda i: (0, i, 0)),
            pl.BlockSpec((2, bt), lambda i: (0, i)),
            full(te_t.shape),
            full(wo1a_t.shape), full(wo1b_t.shape),
            full(wi1a_t.shape), full(wi1b_t.shape),
            full(wo2_t.shape), full(wi2_t.shape),
            full(bo1_t.shape), full(bi1_t.shape),
            full(bo2_t.shape), full(bi2_t.shape),
        ],
        out_specs=pl.BlockSpec((1, 1), lambda i: (0, 0)),
        out_shape=jax.ShapeDtypeStruct((1, 1), jnp.float32),
    )(g, t2, te_t, wo1a_t, wo1b_t, wi1a_t, wi1b_t, wo2_t, wi2_t,
      bo1_t, bi1_t, bo2_t, bi2_t)


def kernel(x, x_t_slot, y, y_t_slot, vecs_use, time_embeddings,
           W_out1, b_out1, W_out2, b_out2, W_in1, b_in1, W_in2, b_in2):
    seq_len, user_len = x.shape
    n = seq_len * user_len
    v, d = vecs_use.shape

    neg_idx = jax.random.randint(jax.random.key(1234), (n,), 0, v, dtype=jnp.int32)
    nh = n // 2
    xf = x.reshape(-1)
    yf = y.reshape(-1)
    xtf = x_t_slot.reshape(-1)
    ytf = y_t_slot.reshape(-1)
    gf = _make_sc_gather(v, d, 3 * nh, 1600)
    weights = (time_embeddings.T,
               W_out1[:d].T, W_out1[d:].T,
               W_in1[:d].T, W_in1[d:].T,
               W_out2.T, W_in2.T,
               b_out1.reshape(d, 1), b_in1.reshape(d, 1),
               b_out2.reshape(d, 1), b_in2.reshape(d, 1))
    loss = jnp.float32(0)
    for sl in (slice(0, nh), slice(nh, None)):
        idx_h = jnp.concatenate([xf[sl], yf[sl], neg_idx[sl]])
        g = gf(vecs_use, idx_h).reshape(3, nh, d)
        t2 = jnp.stack([xtf[sl], ytf[sl]], axis=0)
        loss = loss + _mlp_loss(g, t2, *weights, bt=16384, total_n=n).reshape(())
    return loss
